# Initial kernel scaffold; baseline (speedup 1.0000x reference)
import jax
import jax.numpy as jnp
from jax.experimental import pallas as pl


def kernel(x, edge_index_0, edge_index_1, W1_0, al1_0, ar1_0, b1_0, W1_1, al1_1, ar1_1, b1_1, W2_0, al2_0, ar2_0, b2_0, W2_1, al2_1, ar2_1, b2_1):
    raise NotImplementedError("write your pallas kernel here")


# speedup vs baseline: 1.1656x; 1.1656x over previous
import functools

import jax
import jax.numpy as jnp
from jax.experimental import pallas as pl
from jax.experimental.pallas import tpu as pltpu

N = 10000
E = 320000
IN = 128
HID = 64
OUT = 64
H = 4


def _dense1_body(x_ref, w_ref, al_ref, ar_ref, h_ref, elr_ref):
    x = x_ref[...]
    h = jnp.dot(x, w_ref[...], preferred_element_type=jnp.float32)
    h_ref[...] = h
    bn = x.shape[0]
    h4 = h.reshape(bn, 2 * H, HID)
    el = jnp.sum(h4 * al_ref[...][None], axis=-1)
    er = jnp.sum(h4 * ar_ref[...][None], axis=-1)
    elr_ref[...] = jnp.concatenate([el, er], axis=-1)


def _dense1(x, w, al, ar):
    bn = 1000
    grid = (N // bn,)
    return pl.pallas_call(
        _dense1_body,
        grid=grid,
        in_specs=[
            pl.BlockSpec((bn, IN), lambda i: (i, 0)),
            pl.BlockSpec((IN, 2 * H * HID), lambda i: (0, 0)),
            pl.BlockSpec((2 * H, HID), lambda i: (0, 0)),
            pl.BlockSpec((2 * H, HID), lambda i: (0, 0)),
        ],
        out_specs=[
            pl.BlockSpec((bn, 2 * H * HID), lambda i: (i, 0)),
            pl.BlockSpec((bn, 4 * H), lambda i: (i, 0)),
        ],
        out_shape=[
            jax.ShapeDtypeStruct((N, 2 * H * HID), jnp.float32),
            jax.ShapeDtypeStruct((N, 4 * H), jnp.float32),
        ],
    )(x, w, al, ar)


def _edge_phase(h, el, er, src, dst):
    e = jax.nn.leaky_relu(el[src] + er[dst], negative_slope=0.2)
    ex = jnp.exp(e)
    s = jax.ops.segment_sum(ex, dst, num_segments=N)
    msg = h[src] * ex[..., None]
    acc = jax.ops.segment_sum(msg, dst, num_segments=N)
    return acc / (s[..., None] + 1e-9)


def kernel(x, edge_index_0, edge_index_1,
           W1_0, al1_0, ar1_0, b1_0, W1_1, al1_1, ar1_1, b1_1,
           W2_0, al2_0, ar2_0, b2_0, W2_1, al2_1, ar2_1, b2_1):
    s0, d0 = edge_index_0[0], edge_index_0[1]
    s1, d1 = edge_index_1[0], edge_index_1[1]

    w1 = jnp.concatenate([W1_0, W1_1], axis=1)
    al1 = jnp.concatenate([al1_0, al1_1], axis=0)
    ar1 = jnp.concatenate([ar1_0, ar1_1], axis=0)
    h_all, elr = _dense1(x, w1, al1, ar1)
    h0 = h_all[:, : H * HID].reshape(N, H, HID)
    h1 = h_all[:, H * HID:].reshape(N, H, HID)
    el0, el1 = elr[:, :H], elr[:, H: 2 * H]
    er0, er1 = elr[:, 2 * H: 3 * H], elr[:, 3 * H:]

    a0 = _edge_phase(h0, el0, er0, s0, d0) + b1_0.reshape(1, H, HID)
    a1 = _edge_phase(h1, el1, er1, s1, d1) + b1_1.reshape(1, H, HID)
    hmid = jax.nn.elu((a0 + a1).reshape(N, H * HID))

    w2 = jnp.concatenate([W2_0, W2_1], axis=1)
    al2 = jnp.concatenate([al2_0, al2_1], axis=0).reshape(2, OUT)
    ar2 = jnp.concatenate([ar2_0, ar2_1], axis=0).reshape(2, OUT)
    h2_all, elr2 = _dense2(hmid, w2, al2, ar2)
    g0 = h2_all[:, :OUT].reshape(N, 1, OUT)
    g1 = h2_all[:, OUT:].reshape(N, 1, OUT)
    e20, e21 = elr2[:, :1], elr2[:, 1:2]
    r20, r21 = elr2[:, 2:3], elr2[:, 3:4]

    o0 = _edge_phase(g0, e20, r20, s0, d0) + b2_0.reshape(1, 1, OUT)
    o1 = _edge_phase(g1, e21, r21, s1, d1) + b2_1.reshape(1, 1, OUT)
    return (o0 + o1).reshape(N, OUT)


def _dense2_body(x_ref, w_ref, al_ref, ar_ref, h_ref, elr_ref):
    x = x_ref[...]
    h = jnp.dot(x, w_ref[...], preferred_element_type=jnp.float32)
    h_ref[...] = h
    bn = x.shape[0]
    h2 = h.reshape(bn, 2, OUT)
    el = jnp.sum(h2 * al_ref[...][None], axis=-1)
    er = jnp.sum(h2 * ar_ref[...][None], axis=-1)
    elr_ref[...] = jnp.concatenate([el, er], axis=-1)


def _dense2(x, w, al, ar):
    bn = 1000
    grid = (N // bn,)
    return pl.pallas_call(
        _dense2_body,
        grid=grid,
        in_specs=[
            pl.BlockSpec((bn, H * HID), lambda i: (i, 0)),
            pl.BlockSpec((H * HID, 2 * OUT), lambda i: (0, 0)),
            pl.BlockSpec((2, OUT), lambda i: (0, 0)),
            pl.BlockSpec((2, OUT), lambda i: (0, 0)),
        ],
        out_specs=[
            pl.BlockSpec((bn, 2 * OUT), lambda i: (i, 0)),
            pl.BlockSpec((bn, 4), lambda i: (i, 0)),
        ],
        out_shape=[
            jax.ShapeDtypeStruct((N, 2 * OUT), jnp.float32),
            jax.ShapeDtypeStruct((N, 4), jnp.float32),
        ],
    )(x, w, al, ar)
